# Initial kernel scaffold; baseline (speedup 1.0000x reference)
import jax
import jax.numpy as jnp
from jax.experimental import pallas as pl


def kernel(x, W):
    raise NotImplementedError("write your pallas kernel here")


# speedup vs baseline: 2.0314x; 2.0314x over previous
import functools

import jax
import jax.numpy as jnp
import numpy as np
from jax import lax
from jax.experimental import pallas as pl
from jax.experimental.pallas import tpu as pltpu
from jax.experimental.pallas import tpu_sc as plsc

_NC = 2
_NS = 16
_NW = _NC * _NS


def _get_emb_np(sin_inp):
    emb = np.stack((np.sin(sin_inp), np.cos(sin_inp)), axis=-1)
    return emb.reshape(sin_inp.shape[0], -1)


@functools.lru_cache(maxsize=None)
def _pos_table_np(org_channels, x, y, z):
    channels = int(np.ceil(org_channels / 6) * 2)
    if channels % 2:
        channels += 1
    inv_freq = (1.0 / (10000.0 ** (np.arange(0, channels, 2, dtype=np.float32)
                                   / np.float32(channels)))).astype(np.float32)
    pos_x = np.arange(x, dtype=np.float32)
    pos_y = np.arange(y, dtype=np.float32)
    pos_z = np.arange(z, dtype=np.float32)
    sin_inp_x = np.einsum('i,j->ij', pos_x, inv_freq)
    sin_inp_y = np.einsum('i,j->ij', pos_y, inv_freq)
    sin_inp_z = np.einsum('i,j->ij', pos_z, inv_freq)
    emb_x = np.broadcast_to(_get_emb_np(sin_inp_x)[:, None, None, :],
                            (x, y, z, channels))
    emb_y = np.broadcast_to(_get_emb_np(sin_inp_y)[None, :, None, :],
                            (x, y, z, channels))
    emb_z = np.broadcast_to(_get_emb_np(sin_inp_z)[None, None, :, :],
                            (x, y, z, channels))
    emb = np.concatenate([emb_x, emb_y, emb_z], axis=-1)
    return np.ascontiguousarray(
        emb[:, :, :, :org_channels].reshape(x * y * z, org_channels)
    ).astype(np.float32)


@functools.lru_cache(maxsize=None)
def _build_sc_gather(N, V, D, L):
    per_w = N // _NW
    C = 256
    nchunk = per_w // C
    GSUB = C // 128
    assert per_w % C == 0 and C % 128 == 0 and L % C == 0 and D % 16 == 0

    mesh = plsc.VectorSubcoreMesh(
        core_axis_name="c", subcore_axis_name="s",
        num_cores=_NC, num_subcores=_NS)

    @functools.partial(
        pl.kernel,
        out_type=jax.ShapeDtypeStruct((N, D), jnp.float32),
        mesh=mesh,
        compiler_params=pltpu.CompilerParams(use_tc_tiling_on_sc=False),
        scratch_types=[
            pltpu.VMEM((C,), jnp.int32),
            pltpu.VMEM((C, D), jnp.float32),
            pltpu.VMEM((L, D), jnp.float32),
            pltpu.SemaphoreType.DMA,
        ],
    )
    def body(idx_hbm, pos_hbm, table_hbm, out_hbm, idx_v, rows_v, pos_v, sem):
        cid = lax.axis_index("c")
        sid = lax.axis_index("s")
        wid = sid * _NC + cid
        base = wid * per_w
        pltpu.sync_copy(pos_hbm, pos_v)

        def chunk_body(k, carry):
            g = base + k * C
            pltpu.sync_copy(idx_hbm.at[pl.ds(g, C)], idx_v)
            cps = [
                pltpu.async_copy(table_hbm.at[idx_v.at[pl.ds(j * 128, 128)]],
                                 rows_v.at[pl.ds(j * 128, 128)], sem)
                for j in range(GSUB)
            ]
            for cp in cps:
                cp.wait()
            pos_off = lax.rem(k * C, L)

            def add_row(i, c2):
                p = pos_off + i
                for j in range(D // 16):
                    sl = pl.ds(j * 16, 16)
                    rows_v[i, sl] = rows_v[i, sl] + pos_v[p, sl]
                return c2

            lax.fori_loop(0, C, add_row, 0)
            pltpu.sync_copy(rows_v, out_hbm.at[pl.ds(g, C)])
            return carry

        lax.fori_loop(0, nchunk, chunk_body, 0)

    return body


def kernel(x, W):
    B, L1, L2, orbit = x.shape
    V, D = W.shape
    L = L1 * L2 * orbit
    N = B * L
    pos = jnp.asarray(_pos_table_np(D, L1, L2, orbit))
    flat_idx = x.reshape(N)
    out = _build_sc_gather(N, V, D, L)(flat_idx, pos, W)
    return out.reshape(B, L, D)


# speedup vs baseline: 2.6478x; 1.3034x over previous
import functools

import jax
import jax.numpy as jnp
import numpy as np
from jax import lax
from jax.experimental import pallas as pl
from jax.experimental.pallas import tpu as pltpu
from jax.experimental.pallas import tpu_sc as plsc

_NC = 2
_NS = 16
_NW = _NC * _NS


def _get_emb_np(sin_inp):
    emb = np.stack((np.sin(sin_inp), np.cos(sin_inp)), axis=-1)
    return emb.reshape(sin_inp.shape[0], -1)


@functools.lru_cache(maxsize=None)
def _pos_table_np(org_channels, x, y, z):
    channels = int(np.ceil(org_channels / 6) * 2)
    if channels % 2:
        channels += 1
    inv_freq = (1.0 / (10000.0 ** (np.arange(0, channels, 2, dtype=np.float32)
                                   / np.float32(channels)))).astype(np.float32)
    pos_x = np.arange(x, dtype=np.float32)
    pos_y = np.arange(y, dtype=np.float32)
    pos_z = np.arange(z, dtype=np.float32)
    sin_inp_x = np.einsum('i,j->ij', pos_x, inv_freq)
    sin_inp_y = np.einsum('i,j->ij', pos_y, inv_freq)
    sin_inp_z = np.einsum('i,j->ij', pos_z, inv_freq)
    emb_x = np.broadcast_to(_get_emb_np(sin_inp_x)[:, None, None, :],
                            (x, y, z, channels))
    emb_y = np.broadcast_to(_get_emb_np(sin_inp_y)[None, :, None, :],
                            (x, y, z, channels))
    emb_z = np.broadcast_to(_get_emb_np(sin_inp_z)[None, None, :, :],
                            (x, y, z, channels))
    emb = np.concatenate([emb_x, emb_y, emb_z], axis=-1)
    return np.ascontiguousarray(
        emb[:, :, :, :org_channels].reshape(x * y * z, org_channels)
    ).astype(np.float32)


@functools.lru_cache(maxsize=None)
def _build_sc_gather(N, V, D, L):
    per_w = N // _NW
    C = 256
    nchunk = per_w // C
    GSUB = C // 128
    assert per_w % C == 0 and C % 128 == 0 and L % C == 0 and D % 16 == 0

    mesh = plsc.VectorSubcoreMesh(
        core_axis_name="c", subcore_axis_name="s",
        num_cores=_NC, num_subcores=_NS)

    @functools.partial(
        pl.kernel,
        out_type=jax.ShapeDtypeStruct((N, D), jnp.float32),
        mesh=mesh,
        compiler_params=pltpu.CompilerParams(use_tc_tiling_on_sc=False),
        scratch_types=[
            pltpu.VMEM((C,), jnp.int32),
            pltpu.VMEM((C, D), jnp.float32),
            pltpu.VMEM((L, D), jnp.float32),
            pltpu.SemaphoreType.DMA,
        ],
    )
    def body(idx_hbm, pos_hbm, table_hbm, out_hbm, idx_v, rows_v, pos_v, sem):
        cid = lax.axis_index("c")
        sid = lax.axis_index("s")
        wid = sid * _NC + cid
        base = wid * per_w
        pltpu.sync_copy(pos_hbm, pos_v)

        def chunk_body(k, carry):
            g = base + k * C
            pltpu.sync_copy(idx_hbm.at[pl.ds(g, C)], idx_v)
            pos_off = lax.rem(k * C, L)
            pltpu.sync_copy(pos_hbm.at[pl.ds(pos_off, C)], rows_v)
            cps = [
                pltpu.async_copy(table_hbm.at[idx_v.at[pl.ds(j * 128, 128)]],
                                 rows_v.at[pl.ds(j * 128, 128)], sem,
                                 add=True)
                for j in range(GSUB)
            ]
            for cp in cps:
                cp.wait()
            pltpu.sync_copy(rows_v, out_hbm.at[pl.ds(g, C)])
            return carry

        lax.fori_loop(0, nchunk, chunk_body, 0)

    return body


def kernel(x, W):
    B, L1, L2, orbit = x.shape
    V, D = W.shape
    L = L1 * L2 * orbit
    N = B * L
    pos = jnp.asarray(_pos_table_np(D, L1, L2, orbit))
    flat_idx = x.reshape(N)
    out = _build_sc_gather(N, V, D, L)(flat_idx, pos, W)
    return out.reshape(B, L, D)


# speedup vs baseline: 3.3982x; 1.2834x over previous
import functools

import jax
import jax.numpy as jnp
import numpy as np
from jax import lax
from jax.experimental import pallas as pl
from jax.experimental.pallas import tpu as pltpu
from jax.experimental.pallas import tpu_sc as plsc

_NC = 2
_NS = 16
_NW = _NC * _NS


def _get_emb_np(sin_inp):
    emb = np.stack((np.sin(sin_inp), np.cos(sin_inp)), axis=-1)
    return emb.reshape(sin_inp.shape[0], -1)


@functools.lru_cache(maxsize=None)
def _pos_table_np(org_channels, x, y, z):
    channels = int(np.ceil(org_channels / 6) * 2)
    if channels % 2:
        channels += 1
    inv_freq = (1.0 / (10000.0 ** (np.arange(0, channels, 2, dtype=np.float32)
                                   / np.float32(channels)))).astype(np.float32)
    pos_x = np.arange(x, dtype=np.float32)
    pos_y = np.arange(y, dtype=np.float32)
    pos_z = np.arange(z, dtype=np.float32)
    sin_inp_x = np.einsum('i,j->ij', pos_x, inv_freq)
    sin_inp_y = np.einsum('i,j->ij', pos_y, inv_freq)
    sin_inp_z = np.einsum('i,j->ij', pos_z, inv_freq)
    emb_x = np.broadcast_to(_get_emb_np(sin_inp_x)[:, None, None, :],
                            (x, y, z, channels))
    emb_y = np.broadcast_to(_get_emb_np(sin_inp_y)[None, :, None, :],
                            (x, y, z, channels))
    emb_z = np.broadcast_to(_get_emb_np(sin_inp_z)[None, None, :, :],
                            (x, y, z, channels))
    emb = np.concatenate([emb_x, emb_y, emb_z], axis=-1)
    return np.ascontiguousarray(
        emb[:, :, :, :org_channels].reshape(x * y * z, org_channels)
    ).astype(np.float32)


@functools.lru_cache(maxsize=None)
def _build_sc_gather(N, V, D, L):
    per_w = N // _NW
    C = 512
    nchunk = per_w // C
    GSUB = C // 128
    assert per_w % C == 0 and C % 128 == 0 and L % C == 0
    assert nchunk % 2 == 0

    mesh = plsc.VectorSubcoreMesh(
        core_axis_name="c", subcore_axis_name="s",
        num_cores=_NC, num_subcores=_NS)

    @functools.partial(
        pl.kernel,
        out_type=jax.ShapeDtypeStruct((N, D), jnp.float32),
        mesh=mesh,
        compiler_params=pltpu.CompilerParams(use_tc_tiling_on_sc=False),
        scratch_types=[
            pltpu.VMEM((2, C), jnp.int32),
            pltpu.VMEM((C, D), jnp.float32),
            pltpu.VMEM((C, D), jnp.float32),
            pltpu.VMEM_SHARED((L, D), jnp.float32),
            pltpu.SemaphoreType.DMA,
            pltpu.SemaphoreType.DMA,
            pltpu.SemaphoreType.DMA,
            pltpu.SemaphoreType.DMA,
        ],
    )
    def body(idx_hbm, pos_hbm, table_hbm, out_hbm,
             idx_v, rows0, rows1, pos_sh, g0, g1, w0, w1):
        cid = lax.axis_index("c")
        sid = lax.axis_index("s")
        wid = sid * _NC + cid
        base = wid * per_w
        rows = (rows0, rows1)
        gsem = (g0, g1)
        wsem = (w0, w1)

        @pl.when(sid == 0)
        def _():
            pltpu.sync_copy(pos_hbm, pos_sh)
        plsc.subcore_barrier()

        def fetch_chunk(k, b):
            g = base + k * C
            pltpu.sync_copy(idx_hbm.at[pl.ds(g, C)], idx_v.at[b])
            pos_off = lax.rem(k * C, L)
            pltpu.sync_copy(pos_sh.at[pl.ds(pos_off, C)], rows[b])
            for j in range(GSUB):
                pltpu.async_copy(
                    table_hbm.at[idx_v.at[b].at[pl.ds(j * 128, 128)]],
                    rows[b].at[pl.ds(j * 128, 128)], gsem[b], add=True)

        def wait_gather(b):
            for j in range(GSUB):
                pltpu.make_async_copy(
                    table_hbm.at[idx_v.at[b].at[pl.ds(j * 128, 128)]],
                    rows[b].at[pl.ds(j * 128, 128)], gsem[b]).wait()

        def fire_writeout(k, b):
            g = base + k * C
            pltpu.async_copy(rows[b], out_hbm.at[pl.ds(g, C)], wsem[b])

        def wait_writeout(k, b):
            g = base + k * C
            pltpu.make_async_copy(rows[b], out_hbm.at[pl.ds(g, C)],
                                  wsem[b]).wait()

        fetch_chunk(0, 0)

        def pair_body(i, carry):
            ka = 2 * i

            @pl.when(i > 0)
            def _():
                wait_writeout(ka - 1, 1)
            fetch_chunk(ka + 1, 1)

            wait_gather(0)
            fire_writeout(ka, 0)

            @pl.when(i < nchunk // 2 - 1)
            def _():
                wait_writeout(ka, 0)
                fetch_chunk(ka + 2, 0)

            wait_gather(1)
            fire_writeout(ka + 1, 1)
            return carry

        lax.fori_loop(0, nchunk // 2, pair_body, 0)
        wait_writeout(nchunk - 2, 0)
        wait_writeout(nchunk - 1, 1)

    return body


def kernel(x, W):
    B, L1, L2, orbit = x.shape
    V, D = W.shape
    L = L1 * L2 * orbit
    N = B * L
    pos = jnp.asarray(_pos_table_np(D, L1, L2, orbit))
    flat_idx = x.reshape(N)
    out = _build_sc_gather(N, V, D, L)(flat_idx, pos, W)
    return out.reshape(B, L, D)
